# baseline (speedup 1.0000x reference)
import jax
import jax.numpy as jnp
from jax.experimental import pallas as pl
from jax.experimental.pallas import tpu as pltpu

A = 20000
LANES = 128
ROWS = 160
APAD = ROWS * LANES
NUM_OUT = 500
OUT_ROWS = 512
KEPT_ROWS = 4
THRESH = 0.7
NEG_INF = float("-inf")
NB = 2


def _nms_body(scores_in, anc_ref, del_ref, out_ref, box_ref, sc_ref, kept_ref):
    for b in range(NB):
        ay1 = anc_ref[b, 0]
        ax1 = anc_ref[b, 1]
        ay2 = anc_ref[b, 2]
        ax2 = anc_ref[b, 3]
        h = ay2 - ay1
        w = ax2 - ax1
        cy = ay1 + 0.5 * h
        cx = ax1 + 0.5 * w
        cy = cy + del_ref[b, 0] * h
        cx = cx + del_ref[b, 1] * w
        h = h * jnp.exp(del_ref[b, 2])
        w = w * jnp.exp(del_ref[b, 3])
        y1 = jnp.clip(cy - 0.5 * h, 0.0, 1.0)
        x1 = jnp.clip(cx - 0.5 * w, 0.0, 1.0)
        y2 = jnp.clip(cy + 0.5 * h, 0.0, 1.0)
        x2 = jnp.clip(cx + 0.5 * w, 0.0, 1.0)
        box_ref[b, 0] = y1
        box_ref[b, 1] = x1
        box_ref[b, 2] = y2
        box_ref[b, 3] = x2
        box_ref[b, 4] = (y2 - y1) * (x2 - x1)
        sc_ref[b] = scores_in[b]

    out_ref[...] = jnp.zeros((NB, OUT_ROWS, 4), jnp.float32)
    kept_ref[...] = jnp.zeros((NB, 5, KEPT_ROWS, LANES), jnp.float32)

    iota2d = (jax.lax.broadcasted_iota(jnp.int32, (ROWS, LANES), 0) * LANES
              + jax.lax.broadcasted_iota(jnp.int32, (ROWS, LANES), 1))
    lane_iota = jax.lax.broadcasted_iota(jnp.int32, (1, LANES), 1)

    for b in range(NB):
        def cond(carry):
            kept, done = carry
            return (kept < NUM_OUT) & (done == 0)

        def body(carry):
            kept, done = carry
            scores = sc_ref[b]
            m = jnp.max(scores)
            idx = jnp.min(jnp.where(scores == m, iota2d, APAD))
            valid = m > NEG_INF
            r = idx // LANES
            c = idx % LANES
            lm = lane_iota == c
            sc_ref[b, pl.ds(r, 1), :] = jnp.where(
                lm, NEG_INF, sc_ref[b, pl.ds(r, 1), :])
            by1 = jnp.sum(jnp.where(lm, box_ref[b, 0, pl.ds(r, 1), :], 0.0))
            bx1 = jnp.sum(jnp.where(lm, box_ref[b, 1, pl.ds(r, 1), :], 0.0))
            by2 = jnp.sum(jnp.where(lm, box_ref[b, 2, pl.ds(r, 1), :], 0.0))
            bx2 = jnp.sum(jnp.where(lm, box_ref[b, 3, pl.ds(r, 1), :], 0.0))
            yy1 = jnp.maximum(by1, kept_ref[b, 0])
            xx1 = jnp.maximum(bx1, kept_ref[b, 1])
            yy2 = jnp.minimum(by2, kept_ref[b, 2])
            xx2 = jnp.minimum(bx2, kept_ref[b, 3])
            inter = (jnp.maximum(yy2 - yy1, 0.0)
                     * jnp.maximum(xx2 - xx1, 0.0))
            area_b = (by2 - by1) * (bx2 - bx1)
            union = area_b + kept_ref[b, 4] - inter
            iou = inter / jnp.maximum(union, 1e-12)
            keep = valid & jnp.logical_not(jnp.any(iou > THRESH))

            ks = kept // LANES
            lm2 = lane_iota == (kept % LANES)

            @pl.when(keep)
            def _append():
                kept_ref[b, 0, pl.ds(ks, 1), :] = jnp.where(
                    lm2, by1, kept_ref[b, 0, pl.ds(ks, 1), :])
                kept_ref[b, 1, pl.ds(ks, 1), :] = jnp.where(
                    lm2, bx1, kept_ref[b, 1, pl.ds(ks, 1), :])
                kept_ref[b, 2, pl.ds(ks, 1), :] = jnp.where(
                    lm2, by2, kept_ref[b, 2, pl.ds(ks, 1), :])
                kept_ref[b, 3, pl.ds(ks, 1), :] = jnp.where(
                    lm2, bx2, kept_ref[b, 3, pl.ds(ks, 1), :])
                kept_ref[b, 4, pl.ds(ks, 1), :] = jnp.where(
                    lm2, area_b, kept_ref[b, 4, pl.ds(ks, 1), :])
                row = jnp.concatenate(
                    [by1.reshape(1, 1), bx1.reshape(1, 1),
                     by2.reshape(1, 1), bx2.reshape(1, 1)], axis=1)
                out_ref[b, pl.ds(kept, 1), :] = row

            return (kept + keep.astype(jnp.int32),
                    jnp.logical_not(valid).astype(jnp.int32))

        jax.lax.while_loop(cond, body, (jnp.int32(0), jnp.int32(0)))


@jax.jit
def kernel(rpn_probs, bbox_deltas, anchors):
    B = rpn_probs.shape[0]
    pad = APAD - A
    scores = jnp.pad(rpn_probs[:, :, 1], ((0, 0), (0, pad)),
                     constant_values=NEG_INF).reshape(B, ROWS, LANES)
    anc = jnp.pad(anchors, ((0, 0), (0, pad), (0, 0))).transpose(0, 2, 1)
    anc = anc.reshape(B, 4, ROWS, LANES)
    dlt = jnp.pad(bbox_deltas, ((0, 0), (0, pad), (0, 0))).transpose(0, 2, 1)
    dlt = dlt.reshape(B, 4, ROWS, LANES)

    out = pl.pallas_call(
        _nms_body,
        out_shape=jax.ShapeDtypeStruct((B, OUT_ROWS, 4), jnp.float32),
        scratch_shapes=[
            pltpu.VMEM((NB, 5, ROWS, LANES), jnp.float32),
            pltpu.VMEM((NB, ROWS, LANES), jnp.float32),
            pltpu.VMEM((NB, 5, KEPT_ROWS, LANES), jnp.float32),
        ],
    )(scores, anc, dlt)
    return out[:, :NUM_OUT, :]
